# baseline (speedup 1.0000x reference)
import jax
import jax.numpy as jnp
from jax.experimental import pallas as pl
from jax.experimental.pallas import tpu as pltpu

_BLOCK_N = 512
_BF16 = jnp.bfloat16


def _shift_rows(a, di):
    if di == 0:
        return a
    pad = jnp.zeros((di, a.shape[1]), a.dtype)
    return jnp.concatenate([a[di:], pad], axis=0)


def _fused_kernel(x_ref, m1_ref, b1c_ref, m2_ref, b2c_ref, w1r_ref,
                  b1f_ref, w2p_ref, b2f_ref, w3p_ref, b3f_ref, o_ref,
                  m1c_ref, m2c_ref, w1f_ref, w2b_ref, w3b_ref):
    f32 = jnp.float32

    @pl.when(pl.program_id(0) == 0)
    def _prep():
        m1c_ref[...] = m1_ref[...].astype(_BF16).reshape(160, 256)
        m2c_ref[...] = m2_ref[...].astype(_BF16).reshape(640, 256)
        w1f_ref[0:640, :] = w1r_ref[...].astype(_BF16).reshape(640, 128)
        w1f_ref[640:1024, :] = jnp.zeros((384, 128), _BF16)
        w2b_ref[...] = w2p_ref[...].astype(_BF16)
        w3b_ref[...] = w3p_ref[...].astype(_BF16)

    x4 = x_ref[...]
    M = x4.shape[0] * 32
    x = x4.reshape(M, 32).astype(_BF16)

    xc = jnp.concatenate([_shift_rows(x, di) for di in range(5)], axis=1)
    z1 = jnp.dot(xc, m1c_ref[...], preferred_element_type=f32)
    r1 = jnp.maximum(z1 + b1c_ref[...], 0.0).astype(_BF16)

    h1 = jnp.maximum(r1[:, :128], r1[:, 128:])
    q1 = h1.reshape(M // 2, 256)
    p1 = jnp.maximum(q1[:, :128], q1[:, 128:])

    pc = jnp.concatenate([_shift_rows(p1, di) for di in range(5)], axis=1)
    z2 = jnp.dot(pc, m2c_ref[...], preferred_element_type=f32)
    r2 = jnp.maximum(z2 + b2c_ref[...], 0.0).astype(_BF16)

    h2 = jnp.maximum(r2[:, :128], r2[:, 128:])
    q2 = h2.reshape(M // 4, 256)
    p2 = jnp.maximum(q2[:, :128], q2[:, 128:])

    fv = p2.reshape(M // 32, 1024)
    f = jnp.maximum(jnp.dot(fv, w1f_ref[...], preferred_element_type=f32)
                    + b1f_ref[...], 0.0).astype(_BF16)

    f = jnp.maximum(jnp.dot(f, w2b_ref[...], preferred_element_type=f32)
                    + b2f_ref[...], 0.0).astype(_BF16)
    o = jnp.dot(f, w3b_ref[...], preferred_element_type=f32) + b3f_ref[...]
    o_ref[...] = o[:, :10]


def kernel(m1, b1c, m2, b2c, w1r, b1f, w2p, b2f, w3p, b3f, x_nchw):
    B = x_nchw.shape[0]
    N = _BLOCK_N if B % _BLOCK_N == 0 else B
    M = N * 32

    def full(a):
        if a.ndim == 2:
            return pl.BlockSpec(a.shape, lambda b: (0, 0))
        return pl.BlockSpec(a.shape, lambda b: (0, 0, 0))

    macs_blk = (M * 160 * 256 + (M // 2) * 640 * 256 + N * 1024 * 128
                + 2 * N * 128 * 128)
    out = pl.pallas_call(
        _fused_kernel,
        out_shape=jax.ShapeDtypeStruct((B, 10), jnp.float32),
        grid=(B // N,),
        in_specs=[pl.BlockSpec((N, 1, 32, 32), lambda b: (b, 0, 0, 0)),
                  full(m1), full(b1c), full(m2), full(b2c),
                  full(w1r), full(b1f), full(w2p),
                  full(b2f), full(w3p), full(b3f)],
        out_specs=pl.BlockSpec((N, 10), lambda b: (b, 0)),
        scratch_shapes=[pltpu.VMEM((160, 256), _BF16),
                        pltpu.VMEM((640, 256), _BF16),
                        pltpu.VMEM((1024, 128), _BF16),
                        pltpu.VMEM((128, 128), _BF16),
                        pltpu.VMEM((128, 128), _BF16)],
        compiler_params=pltpu.CompilerParams(
            dimension_semantics=("arbitrary",),
            vmem_limit_bytes=64 * 1024 * 1024),
        cost_estimate=pl.CostEstimate(
            flops=2 * macs_blk * (B // N), transcendentals=0,
            bytes_accessed=4 * B * 32 * 32 + 4 * B * 10),
    )(x_nchw, m1, b1c, m2, b2c, w1r, b1f, w2p, b2f, w3p, b3f)
    return out


# speedup vs baseline: 1.4356x; 1.4356x over previous
import jax
import jax.numpy as jnp
from jax.experimental import pallas as pl
from jax.experimental.pallas import tpu as pltpu

_BLOCK_N = 512
_BF16 = jnp.bfloat16


def _shift_rows(a, di):
    if di == 0:
        return a
    pad = jnp.zeros((di, a.shape[1]), a.dtype)
    return jnp.concatenate([a[di:], pad], axis=0)


def _plane_kernel(x_ref, m1_ref, m2_ref, w1r_ref, b1c_ref, b2c_ref,
                  b1f_ref, w2p_ref, b2f_ref, w3p_ref, b3f_ref, o_ref,
                  wa_ref, wb_ref, va_ref, vb_ref, w1q_ref, w2b_ref,
                  w3b_ref):
    f32 = jnp.float32

    @pl.when(pl.program_id(0) == 0)
    def _prep():
        wa_ref[...] = jnp.zeros_like(wa_ref)
        wb_ref[...] = jnp.zeros_like(wb_ref)
        va_ref[...] = jnp.zeros_like(va_ref)
        vb_ref[...] = jnp.zeros_like(vb_ref)
        w1q_ref[...] = jnp.zeros_like(w1q_ref)
        m1 = m1_ref[...].astype(_BF16)
        m2 = m2_ref[...].astype(_BF16)
        w1r = w1r_ref[...].astype(_BF16)
        for s in range(8):
            for di in range(5):
                r = s + di
                if r <= 7:
                    wa_ref[s, r * 32:(r + 1) * 32, :] = m1[di]
                else:
                    wb_ref[s - 4, (r - 8) * 32:(r - 7) * 32, :] = m1[di]
        for u in range(4):
            for di in range(5):
                t = u + di
                if t <= 3:
                    va_ref[u, t * 128:(t + 1) * 128, :] = m2[di]
                else:
                    vb_ref[u, (t - 4) * 128:(t - 3) * 128, :] = m2[di]
        for g in range(3):
            for v in range(2):
                if 2 * g + v <= 4:
                    w1q_ref[g, v * 128:(v + 1) * 128, :] = w1r[2 * g + v]
        w2b_ref[...] = w2p_ref[...].astype(_BF16)
        w3b_ref[...] = w3p_ref[...].astype(_BF16)

    xr = x_ref[...].astype(_BF16)
    G = xr.shape[0] * 4
    xw = xr.reshape(G, 256)
    xw1 = _shift_rows(xw, 1)

    def conv1_plane(s):
        z = jnp.dot(xw, wa_ref[s], preferred_element_type=f32)
        if s >= 4:
            z = z + jnp.dot(xw1, wb_ref[s - 4], preferred_element_type=f32)
        r = jnp.maximum(z + b1c_ref[...], 0.0)
        return jnp.maximum(r[:, :128], r[:, 128:])

    w1 = [conv1_plane(s) for s in range(8)]
    p1 = [jnp.maximum(w1[2 * t], w1[2 * t + 1]).astype(_BF16)
          for t in range(4)]

    P = jnp.concatenate(p1, axis=1)
    P1 = _shift_rows(P, 1)

    def conv2_plane(u):
        z = (jnp.dot(P, va_ref[u], preferred_element_type=f32)
             + jnp.dot(P1, vb_ref[u], preferred_element_type=f32))
        r = jnp.maximum(z + b2c_ref[...], 0.0)
        return jnp.maximum(r[:, :128], r[:, 128:])

    w2 = [conv2_plane(u) for u in range(4)]
    p2 = [jnp.maximum(w2[0], w2[1]).astype(_BF16),
          jnp.maximum(w2[2], w2[3]).astype(_BF16)]

    Q = jnp.concatenate(p2, axis=1)
    F = jnp.dot(Q, w1q_ref[0], preferred_element_type=f32)
    for g in range(1, 3):
        F = F + jnp.dot(_shift_rows(Q, g), w1q_ref[g],
                        preferred_element_type=f32)
    F = jnp.maximum(F + b1f_ref[...], 0.0).astype(_BF16)

    F = jnp.maximum(jnp.dot(F, w2b_ref[...], preferred_element_type=f32)
                    + b2f_ref[...], 0.0).astype(_BF16)
    F = jnp.dot(F, w3b_ref[...], preferred_element_type=f32) + b3f_ref[...]

    o_ref[...] = F.reshape(G // 4, 512)[:, :10]


def kernel(m1, b1c, m2, b2c, w1r, b1f, w2p, b2f, w3p, b3f, x_nchw):
    B = x_nchw.shape[0]
    N = _BLOCK_N if B % _BLOCK_N == 0 else B
    G = N * 4
    xf = x_nchw.reshape(B, 1024)

    def full(a):
        if a.ndim == 2:
            return pl.BlockSpec(a.shape, lambda b: (0, 0))
        return pl.BlockSpec(a.shape, lambda b: (0, 0, 0))

    macs_blk = (12 * G * 256 * 256 + 8 * G * 512 * 256 + 3 * G * 256 * 128
                + 2 * G * 128 * 128)
    out = pl.pallas_call(
        _plane_kernel,
        out_shape=jax.ShapeDtypeStruct((B, 10), jnp.float32),
        grid=(B // N,),
        in_specs=[pl.BlockSpec((N, 1024), lambda b: (b, 0)),
                  full(m1), full(m2), full(w1r), full(b1c), full(b2c),
                  full(b1f), full(w2p), full(b2f), full(w3p), full(b3f)],
        out_specs=pl.BlockSpec((N, 10), lambda b: (b, 0)),
        scratch_shapes=[pltpu.VMEM((8, 256, 256), _BF16),
                        pltpu.VMEM((4, 256, 256), _BF16),
                        pltpu.VMEM((4, 512, 256), _BF16),
                        pltpu.VMEM((4, 512, 256), _BF16),
                        pltpu.VMEM((3, 256, 128), _BF16),
                        pltpu.VMEM((128, 128), _BF16),
                        pltpu.VMEM((128, 128), _BF16)],
        compiler_params=pltpu.CompilerParams(
            dimension_semantics=("arbitrary",),
            vmem_limit_bytes=64 * 1024 * 1024),
        cost_estimate=pl.CostEstimate(
            flops=2 * macs_blk * (B // N), transcendentals=0,
            bytes_accessed=4 * B * 32 * 32 + 4 * B * 10),
    )(xf, m1, m2, w1r, b1c, b2c, b1f, w2p, b2f, w3p, b3f)
    return out
